# Initial kernel scaffold; baseline (speedup 1.0000x reference)
import jax
import jax.numpy as jnp
from jax import lax
from jax.experimental import pallas as pl
from jax.experimental.pallas import tpu as pltpu
from jax.experimental.pallas import tpu_sc as plsc

_N, _D, _H, _E, _T = 10000, 128, 64, 320000, 10
_NC, _NS = 2, 16
_NW = _NC * _NS
_EPW = _E // _NW
_C = 80
_NCH = _EPW // _C
_RPS = _N // _NS
_ZR = 125
_NP1 = 10240
_CPS = _NP1 // _NS

_R = 1000
_G = _N // _R


def _segsum_body(y_hbm, src_hbm, dst_hbm, out_hbm,
                 src_v, dst_v, rows_v, tmp_v, acc_sh, sem):
    c = lax.axis_index("c")
    s = lax.axis_index("s")
    wid = s * _NC + c

    def zrow(i, carry):
        for j in range(_H // 16):
            tmp_v[i, pl.ds(j * 16, 16)] = jnp.zeros((16,), jnp.float32)
        return carry
    lax.fori_loop(0, _ZR, zrow, 0)
    row0 = s * _RPS
    for k in range(_RPS // _ZR):
        pltpu.sync_copy(tmp_v, acc_sh.at[pl.ds(row0 + k * _ZR, _ZR)])

    pltpu.sync_copy(src_hbm.at[wid], src_v)
    pltpu.sync_copy(dst_hbm.at[wid], dst_v)
    plsc.subcore_barrier()

    def chunk(j, carry):
        pltpu.async_copy(y_hbm.at[src_v.at[j]], rows_v, sem).wait()
        pltpu.sync_copy(rows_v, acc_sh.at[dst_v.at[j]], add=True)
        return carry
    lax.fori_loop(0, _NCH, chunk, 0)
    plsc.subcore_barrier()

    for k in range(_RPS // _ZR):
        r = row0 + k * _ZR
        pltpu.sync_copy(acc_sh.at[pl.ds(r, _ZR)], tmp_v)
        pltpu.sync_copy(tmp_v, out_hbm.at[c, pl.ds(r, _ZR)])


def _make_segsum():
    mesh = plsc.VectorSubcoreMesh(core_axis_name="c", subcore_axis_name="s")
    return pl.kernel(
        _segsum_body,
        out_type=jax.ShapeDtypeStruct((_NC, _N, _H), jnp.float32),
        mesh=mesh,
        scratch_types=[
            pltpu.VMEM((_NCH, _C), jnp.int32),
            pltpu.VMEM((_NCH, _C), jnp.int32),
            pltpu.VMEM((_C, _H), jnp.float32),
            pltpu.VMEM((_ZR, _H), jnp.float32),
            pltpu.VMEM_SHARED((_N, _H), jnp.float32),
            pltpu.SemaphoreType.DMA,
        ],
    )


def _cnt_body(dst_hbm, out_hbm, dst_v, ones_v, tmp_v, acc_sh):
    c = lax.axis_index("c")
    s = lax.axis_index("s")
    wid = s * _NC + c

    for j in range(_CPS // 16):
        tmp_v[pl.ds(j * 16, 16)] = jnp.zeros((16,), jnp.float32)
    for j in range(_C // 16):
        ones_v[pl.ds(j * 16, 16)] = jnp.ones((16,), jnp.float32)
    pltpu.sync_copy(tmp_v, acc_sh.at[pl.ds(s * _CPS, _CPS)])
    pltpu.sync_copy(dst_hbm.at[wid], dst_v)
    plsc.subcore_barrier()

    def chunk(j, carry):
        pltpu.sync_copy(ones_v, acc_sh.at[dst_v.at[j]], add=True)
        return carry
    lax.fori_loop(0, _NCH, chunk, 0)
    plsc.subcore_barrier()

    pltpu.sync_copy(acc_sh.at[pl.ds(s * _CPS, _CPS)], tmp_v)
    pltpu.sync_copy(tmp_v, out_hbm.at[c, pl.ds(s * _CPS, _CPS)])


def _make_cnt():
    mesh = plsc.VectorSubcoreMesh(core_axis_name="c", subcore_axis_name="s")
    return pl.kernel(
        _cnt_body,
        out_type=jax.ShapeDtypeStruct((_NC, _NP1), jnp.float32),
        mesh=mesh,
        scratch_types=[
            pltpu.VMEM((_NCH, _C), jnp.int32),
            pltpu.VMEM((_C,), jnp.float32),
            pltpu.VMEM((_CPS,), jnp.float32),
            pltpu.VMEM_SHARED((_NP1,), jnp.float32),
        ],
    )


def _full(shape):
    return pl.BlockSpec(shape, lambda i: tuple(0 for _ in shape))


def _pre_body(x_ref, wl_ref, wr_ref, b_ref, y_ref, r_ref):
    x = x_ref[...]
    y_ref[...] = jnp.dot(x, wl_ref[...], preferred_element_type=jnp.float32)
    r_ref[...] = jnp.dot(x, wr_ref[...], preferred_element_type=jnp.float32) + b_ref[...]


def _tc_pre(x, wl, wr, b):
    return pl.pallas_call(
        _pre_body,
        grid=(_G,),
        in_specs=[pl.BlockSpec((_R, _D), lambda i: (i, 0)),
                  _full((_D, _H)), _full((_D, _H)), _full((1, _H))],
        out_specs=[pl.BlockSpec((_R, _H), lambda i: (i, 0))] * 2,
        out_shape=[jax.ShapeDtypeStruct((_N, _H), jnp.float32)] * 2,
    )(x, wl, wr, b)


def _mid_body(p_ref, inv_ref, r_ref, wl_ref, wr_ref, b_ref, y_ref, r2_ref):
    agg = (p_ref[0] + p_ref[1]) * inv_ref[...]
    h = jnp.maximum(agg + r_ref[...], 0.0)
    y_ref[...] = jnp.dot(h, wl_ref[...], preferred_element_type=jnp.float32)
    r2_ref[...] = jnp.dot(h, wr_ref[...], preferred_element_type=jnp.float32) + b_ref[...]


def _tc_mid(p, inv, r, wl, wr, b):
    return pl.pallas_call(
        _mid_body,
        grid=(_G,),
        in_specs=[pl.BlockSpec((2, _R, _H), lambda i: (0, i, 0)),
                  pl.BlockSpec((_R, 1), lambda i: (i, 0)),
                  pl.BlockSpec((_R, _H), lambda i: (i, 0)),
                  _full((_H, _H)), _full((_H, _H)), _full((1, _H))],
        out_specs=[pl.BlockSpec((_R, _H), lambda i: (i, 0))] * 2,
        out_shape=[jax.ShapeDtypeStruct((_N, _H), jnp.float32)] * 2,
    )(p, inv, r, wl, wr, b)


def _mid2_body(p_ref, inv_ref, r_ref, wr3_ref, b3_ref, h_ref, rr_ref):
    agg = (p_ref[0] + p_ref[1]) * inv_ref[...]
    h = jnp.maximum(agg + r_ref[...], 0.0)
    h_ref[...] = h
    rr_ref[...] = jnp.dot(h, wr3_ref[...], preferred_element_type=jnp.float32) + b3_ref[...]


def _tc_mid2(p, inv, r, wr3, b3):
    return pl.pallas_call(
        _mid2_body,
        grid=(_G,),
        in_specs=[pl.BlockSpec((2, _R, _H), lambda i: (0, i, 0)),
                  pl.BlockSpec((_R, 1), lambda i: (i, 0)),
                  pl.BlockSpec((_R, _H), lambda i: (i, 0)),
                  _full((_H, _D)), _full((1, _D))],
        out_specs=[pl.BlockSpec((_R, _H), lambda i: (i, 0)),
                   pl.BlockSpec((_R, _D), lambda i: (i, 0))],
        out_shape=[jax.ShapeDtypeStruct((_N, _H), jnp.float32),
                   jax.ShapeDtypeStruct((_N, _D), jnp.float32)],
    )(p, inv, r, wr3, b3)


def _step_body(p_ref, inv_ref, rr_ref, xt_ref, dt_ref, wl3_ref,
               wl1_ref, wr1_ref, b1_ref, xn_ref, y_ref, r_ref):
    a3 = (p_ref[0] + p_ref[1]) * inv_ref[...]
    dx = jnp.dot(a3, wl3_ref[...], preferred_element_type=jnp.float32) + rr_ref[...]
    xn = xt_ref[...] + dt_ref[...] * dx
    xn_ref[...] = xn
    y_ref[...] = jnp.dot(xn, wl1_ref[...], preferred_element_type=jnp.float32)
    r_ref[...] = jnp.dot(xn, wr1_ref[...], preferred_element_type=jnp.float32) + b1_ref[...]


def _tc_step(p, inv, rr, xt, dt, wl3, wl1, wr1, b1):
    return pl.pallas_call(
        _step_body,
        grid=(_G,),
        in_specs=[pl.BlockSpec((2, _R, _H), lambda i: (0, i, 0)),
                  pl.BlockSpec((_R, 1), lambda i: (i, 0)),
                  pl.BlockSpec((_R, _D), lambda i: (i, 0)),
                  pl.BlockSpec((_R, _D), lambda i: (i, 0)),
                  _full((1, 1)), _full((_H, _D)),
                  _full((_D, _H)), _full((_D, _H)), _full((1, _H))],
        out_specs=[pl.BlockSpec((_R, _D), lambda i: (i, 0)),
                   pl.BlockSpec((_R, _H), lambda i: (i, 0)),
                   pl.BlockSpec((_R, _H), lambda i: (i, 0))],
        out_shape=[jax.ShapeDtypeStruct((_N, _D), jnp.float32),
                   jax.ShapeDtypeStruct((_N, _H), jnp.float32),
                   jax.ShapeDtypeStruct((_N, _H), jnp.float32)],
    )(p, inv, rr, xt, dt, wl3, wl1, wr1, b1)


def _dec_body(s_ref, w_ref, b_ref, o_ref):
    o_ref[...] = (jnp.dot(s_ref[0], w_ref[...], preferred_element_type=jnp.float32)
                  + b_ref[...])[None]


def _tc_dec(sol, w, b):
    return pl.pallas_call(
        _dec_body,
        grid=(_T, _G),
        in_specs=[pl.BlockSpec((1, _R, _D), lambda t, i: (t, i, 0)),
                  pl.BlockSpec((_D, 2), lambda t, i: (0, 0)),
                  pl.BlockSpec((1, 2), lambda t, i: (0, 0))],
        out_specs=pl.BlockSpec((1, _R, 2), lambda t, i: (t, i, 0)),
        out_shape=jax.ShapeDtypeStruct((_T, _N, 2), jnp.float32),
    )(sol, w, b)


def kernel(x, edge_index, time_span, Wl1, Wr1, b1, Wl2, Wr2, b2,
           Wl3, Wr3, b3, Wdec, bdec):
    src3 = edge_index[0].reshape(_NW, _NCH, _C)
    dst3 = edge_index[1].reshape(_NW, _NCH, _C)

    segsum = _make_segsum()
    cntk = _make_cnt()

    cntp = cntk(dst3)
    cnt = cntp[0, :_N] + cntp[1, :_N]
    inv = (1.0 / jnp.maximum(cnt, 1.0))[:, None]

    b1r = b1.reshape(1, _H)
    b2r = b2.reshape(1, _H)
    b3r = b3.reshape(1, _D)
    bdr = bdec.reshape(1, 2)
    dts = (time_span[1:] - time_span[:-1]).reshape(_T - 1, 1, 1)

    y1, r1 = _tc_pre(x, Wl1, Wr1, b1r)
    sols = [x]
    xt = x
    for i in range(1, _T):
        p1 = segsum(y1, src3, dst3)
        y2, r2 = _tc_mid(p1, inv, r1, Wl2, Wr2, b2r)
        p2 = segsum(y2, src3, dst3)
        h2, rr3 = _tc_mid2(p2, inv, r2, Wr3, b3r)
        p3 = segsum(h2, src3, dst3)
        xt, y1, r1 = _tc_step(p3, inv, rr3, xt, dts[i - 1], Wl3, Wl1, Wr1, b1r)
        sols.append(xt)

    solution = jnp.stack(sols, axis=0)
    trajectories = _tc_dec(solution, Wdec, bdr)
    return trajectories, solution


# speedup vs baseline: 7.4563x; 7.4563x over previous
import jax
import jax.numpy as jnp
from jax import lax
from jax.experimental import pallas as pl
from jax.experimental.pallas import tpu as pltpu
from jax.experimental.pallas import tpu_sc as plsc

_N, _D, _H, _E, _T = 10000, 128, 64, 320000, 10
_NC, _NS = 2, 16
_NW = _NC * _NS
_EPW = _E // _NW
_C = 80
_NCH = _EPW // _C
_NP = 10240
_RPS = _NP // _NS
_ZR = 128
_NP1 = 10240
_CPS = _NP1 // _NS

_R = 1000
_G = _N // _R


def _segsum_body(y_hbm, src_hbm, dst_hbm, out_hbm,
                 src_v, dst_v, rows_v, tmp_v, acc_sh, sem):
    c = lax.axis_index("c")
    s = lax.axis_index("s")
    wid = s * _NC + c

    def zrow(i, carry):
        for j in range(_H // 16):
            tmp_v[i, pl.ds(j * 16, 16)] = jnp.zeros((16,), jnp.float32)
        return carry
    lax.fori_loop(0, _ZR, zrow, 0)
    row0 = s * _RPS
    for k in range(_RPS // _ZR):
        pltpu.sync_copy(tmp_v, acc_sh.at[pl.ds(row0 + k * _ZR, _ZR)])

    pltpu.sync_copy(src_hbm.at[wid], src_v)
    pltpu.sync_copy(dst_hbm.at[wid], dst_v)
    plsc.subcore_barrier()

    def chunk(j, carry):
        pltpu.async_copy(y_hbm.at[src_v.at[j]], rows_v, sem).wait()
        pltpu.sync_copy(rows_v, acc_sh.at[dst_v.at[j]], add=True)
        return carry
    lax.fori_loop(0, _NCH, chunk, 0)
    plsc.subcore_barrier()

    for k in range(_RPS // _ZR):
        r = row0 + k * _ZR
        pltpu.sync_copy(acc_sh.at[pl.ds(r, _ZR)], tmp_v)
        pltpu.sync_copy(tmp_v, out_hbm.at[c, pl.ds(r, _ZR)])


def _make_segsum():
    mesh = plsc.VectorSubcoreMesh(core_axis_name="c", subcore_axis_name="s",
                                  num_cores=_NC, num_subcores=_NS)
    return pl.kernel(
        _segsum_body,
        out_type=jax.ShapeDtypeStruct((_NC, _NP, _H), jnp.float32),
        mesh=mesh,
        scratch_types=[
            pltpu.VMEM((_NCH, _C), jnp.int32),
            pltpu.VMEM((_NCH, _C), jnp.int32),
            pltpu.VMEM((_C, _H), jnp.float32),
            pltpu.VMEM((_ZR, _H), jnp.float32),
            pltpu.VMEM_SHARED((_NP, _H), jnp.float32),
            pltpu.SemaphoreType.DMA,
        ],
        compiler_params=pltpu.CompilerParams(use_tc_tiling_on_sc=False),
    )


def _cnt_body(dst_hbm, out_hbm, dst_v, ones_v, tmp_v, acc_sh):
    c = lax.axis_index("c")
    s = lax.axis_index("s")
    wid = s * _NC + c

    for j in range(_CPS // 16):
        tmp_v[pl.ds(j * 16, 16)] = jnp.zeros((16,), jnp.float32)
    for j in range(_C // 16):
        ones_v[pl.ds(j * 16, 16)] = jnp.ones((16,), jnp.float32)
    pltpu.sync_copy(tmp_v, acc_sh.at[pl.ds(s * _CPS, _CPS)])
    pltpu.sync_copy(dst_hbm.at[wid], dst_v)
    plsc.subcore_barrier()

    def chunk(j, carry):
        pltpu.sync_copy(ones_v, acc_sh.at[dst_v.at[j]], add=True)
        return carry
    lax.fori_loop(0, _NCH, chunk, 0)
    plsc.subcore_barrier()

    pltpu.sync_copy(acc_sh.at[pl.ds(s * _CPS, _CPS)], tmp_v)
    pltpu.sync_copy(tmp_v, out_hbm.at[c, pl.ds(s * _CPS, _CPS)])


def _make_cnt():
    mesh = plsc.VectorSubcoreMesh(core_axis_name="c", subcore_axis_name="s",
                                  num_cores=_NC, num_subcores=_NS)
    return pl.kernel(
        _cnt_body,
        out_type=jax.ShapeDtypeStruct((_NC, _NP1), jnp.float32),
        mesh=mesh,
        scratch_types=[
            pltpu.VMEM((_NCH, _C), jnp.int32),
            pltpu.VMEM((_C,), jnp.float32),
            pltpu.VMEM((_CPS,), jnp.float32),
            pltpu.VMEM_SHARED((_NP1,), jnp.float32),
        ],
        compiler_params=pltpu.CompilerParams(use_tc_tiling_on_sc=False),
    )


def _full(shape):
    return pl.BlockSpec(shape, lambda i: tuple(0 for _ in shape))


def _pre_body(x_ref, wl_ref, wr_ref, b_ref, y_ref, r_ref):
    x = x_ref[...]
    y_ref[...] = jnp.dot(x, wl_ref[...], preferred_element_type=jnp.float32)
    r_ref[...] = jnp.dot(x, wr_ref[...], preferred_element_type=jnp.float32) + b_ref[...]


def _tc_pre(x, wl, wr, b):
    return pl.pallas_call(
        _pre_body,
        grid=(_G,),
        in_specs=[pl.BlockSpec((_R, _D), lambda i: (i, 0)),
                  _full((_D, _H)), _full((_D, _H)), _full((1, _H))],
        out_specs=[pl.BlockSpec((_R, _H), lambda i: (i, 0))] * 2,
        out_shape=[jax.ShapeDtypeStruct((_N, _H), jnp.float32)] * 2,
    )(x, wl, wr, b)


def _mid_body(p_ref, inv_ref, r_ref, wl_ref, wr_ref, b_ref, y_ref, r2_ref):
    agg = (p_ref[0] + p_ref[1]) * inv_ref[...]
    h = jnp.maximum(agg + r_ref[...], 0.0)
    y_ref[...] = jnp.dot(h, wl_ref[...], preferred_element_type=jnp.float32)
    r2_ref[...] = jnp.dot(h, wr_ref[...], preferred_element_type=jnp.float32) + b_ref[...]


def _tc_mid(p, inv, r, wl, wr, b):
    return pl.pallas_call(
        _mid_body,
        grid=(_G,),
        in_specs=[pl.BlockSpec((2, _R, _H), lambda i: (0, i, 0)),
                  pl.BlockSpec((_R, 1), lambda i: (i, 0)),
                  pl.BlockSpec((_R, _H), lambda i: (i, 0)),
                  _full((_H, _H)), _full((_H, _H)), _full((1, _H))],
        out_specs=[pl.BlockSpec((_R, _H), lambda i: (i, 0))] * 2,
        out_shape=[jax.ShapeDtypeStruct((_N, _H), jnp.float32)] * 2,
    )(p, inv, r, wl, wr, b)


def _mid2_body(p_ref, inv_ref, r_ref, wr3_ref, b3_ref, h_ref, rr_ref):
    agg = (p_ref[0] + p_ref[1]) * inv_ref[...]
    h = jnp.maximum(agg + r_ref[...], 0.0)
    h_ref[...] = h
    rr_ref[...] = jnp.dot(h, wr3_ref[...], preferred_element_type=jnp.float32) + b3_ref[...]


def _tc_mid2(p, inv, r, wr3, b3):
    return pl.pallas_call(
        _mid2_body,
        grid=(_G,),
        in_specs=[pl.BlockSpec((2, _R, _H), lambda i: (0, i, 0)),
                  pl.BlockSpec((_R, 1), lambda i: (i, 0)),
                  pl.BlockSpec((_R, _H), lambda i: (i, 0)),
                  _full((_H, _D)), _full((1, _D))],
        out_specs=[pl.BlockSpec((_R, _H), lambda i: (i, 0)),
                   pl.BlockSpec((_R, _D), lambda i: (i, 0))],
        out_shape=[jax.ShapeDtypeStruct((_N, _H), jnp.float32),
                   jax.ShapeDtypeStruct((_N, _D), jnp.float32)],
    )(p, inv, r, wr3, b3)


def _step_body(p_ref, inv_ref, rr_ref, xt_ref, dt_ref, wl3_ref,
               wl1_ref, wr1_ref, b1_ref, xn_ref, y_ref, r_ref):
    a3 = (p_ref[0] + p_ref[1]) * inv_ref[...]
    dx = jnp.dot(a3, wl3_ref[...], preferred_element_type=jnp.float32) + rr_ref[...]
    xn = xt_ref[...] + dt_ref[...] * dx
    xn_ref[...] = xn
    y_ref[...] = jnp.dot(xn, wl1_ref[...], preferred_element_type=jnp.float32)
    r_ref[...] = jnp.dot(xn, wr1_ref[...], preferred_element_type=jnp.float32) + b1_ref[...]


def _tc_step(p, inv, rr, xt, dt, wl3, wl1, wr1, b1):
    return pl.pallas_call(
        _step_body,
        grid=(_G,),
        in_specs=[pl.BlockSpec((2, _R, _H), lambda i: (0, i, 0)),
                  pl.BlockSpec((_R, 1), lambda i: (i, 0)),
                  pl.BlockSpec((_R, _D), lambda i: (i, 0)),
                  pl.BlockSpec((_R, _D), lambda i: (i, 0)),
                  _full((1, 1)), _full((_H, _D)),
                  _full((_D, _H)), _full((_D, _H)), _full((1, _H))],
        out_specs=[pl.BlockSpec((_R, _D), lambda i: (i, 0)),
                   pl.BlockSpec((_R, _H), lambda i: (i, 0)),
                   pl.BlockSpec((_R, _H), lambda i: (i, 0))],
        out_shape=[jax.ShapeDtypeStruct((_N, _D), jnp.float32),
                   jax.ShapeDtypeStruct((_N, _H), jnp.float32),
                   jax.ShapeDtypeStruct((_N, _H), jnp.float32)],
    )(p, inv, rr, xt, dt, wl3, wl1, wr1, b1)


def _dec_body(s_ref, w_ref, b_ref, o_ref):
    o_ref[...] = (jnp.dot(s_ref[0], w_ref[...], preferred_element_type=jnp.float32)
                  + b_ref[...])[None]


def _tc_dec(sol, w, b):
    return pl.pallas_call(
        _dec_body,
        grid=(_T, _G),
        in_specs=[pl.BlockSpec((1, _R, _D), lambda t, i: (t, i, 0)),
                  pl.BlockSpec((_D, 2), lambda t, i: (0, 0)),
                  pl.BlockSpec((1, 2), lambda t, i: (0, 0))],
        out_specs=pl.BlockSpec((1, _R, 2), lambda t, i: (t, i, 0)),
        out_shape=jax.ShapeDtypeStruct((_T, _N, 2), jnp.float32),
    )(sol, w, b)


def kernel(x, edge_index, time_span, Wl1, Wr1, b1, Wl2, Wr2, b2,
           Wl3, Wr3, b3, Wdec, bdec):
    src3 = edge_index[0].reshape(_NW, _NCH, _C)
    dst3 = edge_index[1].reshape(_NW, _NCH, _C)

    segsum = _make_segsum()
    cntk = _make_cnt()

    cntp = cntk(dst3)
    cnt = cntp[0, :_N] + cntp[1, :_N]
    inv = (1.0 / jnp.maximum(cnt, 1.0))[:, None]

    b1r = b1.reshape(1, _H)
    b2r = b2.reshape(1, _H)
    b3r = b3.reshape(1, _D)
    bdr = bdec.reshape(1, 2)
    dts = (time_span[1:] - time_span[:-1]).reshape(_T - 1, 1, 1)

    y1, r1 = _tc_pre(x, Wl1, Wr1, b1r)
    sols = [x]
    xt = x
    for i in range(1, _T):
        p1 = segsum(y1, src3, dst3)
        y2, r2 = _tc_mid(p1, inv, r1, Wl2, Wr2, b2r)
        p2 = segsum(y2, src3, dst3)
        h2, rr3 = _tc_mid2(p2, inv, r2, Wr3, b3r)
        p3 = segsum(h2, src3, dst3)
        xt, y1, r1 = _tc_step(p3, inv, rr3, xt, dts[i - 1], Wl3, Wl1, Wr1, b1r)
        sols.append(xt)

    solution = jnp.stack(sols, axis=0)
    trajectories = _tc_dec(solution, Wdec, bdr)
    return trajectories, solution


# speedup vs baseline: 13.0596x; 1.7515x over previous
import jax
import jax.numpy as jnp
from jax import lax
from jax.experimental import pallas as pl
from jax.experimental.pallas import tpu as pltpu
from jax.experimental.pallas import tpu_sc as plsc

_N, _D, _H, _E, _T = 10000, 128, 64, 320000, 10
_NC, _NS = 2, 16
_NW = _NC * _NS
_C = 128
_EPW = 10240
_EP = _NW * _EPW
_NCH = _EPW // _C
_NB = 2
_NG = _NCH // _NB
_NP = 10240
_RPS = _NP // _NS
_ZR = 128
_NP1 = 10240
_CPS = _NP1 // _NS

_R = 1000
_G = _N // _R


def _segsum_body(y_hbm, src_hbm, dst_hbm, out_hbm,
                 src_v, dst_v, rows0_v, rows1_v, acc_sh,
                 gsem, ssem0, ssem1):
    c = lax.axis_index("c")
    s = lax.axis_index("s")
    wid = s * _NC + c
    rows = (rows0_v, rows1_v)
    ssems = (ssem0, ssem1)
    tmp_v = rows0_v.at[0]

    def zrow(i, carry):
        for j in range(_H // 16):
            tmp_v[i, pl.ds(j * 16, 16)] = jnp.zeros((16,), jnp.float32)
        return carry
    lax.fori_loop(0, _ZR, zrow, 0)
    row0 = s * _RPS
    for k in range(_RPS // _ZR):
        pltpu.sync_copy(tmp_v, acc_sh.at[pl.ds(row0 + k * _ZR, _ZR)])

    pltpu.sync_copy(src_hbm.at[wid], src_v)
    pltpu.sync_copy(dst_hbm.at[wid], dst_v)
    plsc.subcore_barrier()

    def issue_g(g, bs):
        return [pltpu.async_copy(y_hbm.at[src_v.at[g * _NB + b]],
                                 rows[bs].at[b], gsem) for b in range(_NB)]

    def wait_g(descs):
        for d in descs:
            d.wait()

    def issue_s(g, bs):
        return [pltpu.async_copy(rows[bs].at[b],
                                 acc_sh.at[dst_v.at[g * _NB + b]], ssems[bs],
                                 add=True) for b in range(_NB)]

    wait_g(issue_g(0, 0))

    def pair(k, carry):
        g0 = 2 * k
        sd0 = issue_s(g0, 0)
        gd1 = issue_g(g0 + 1, 1)
        wait_g(gd1)
        sd1 = issue_s(g0 + 1, 1)
        for d in sd0:
            d.wait()

        @pl.when(k < _NG // 2 - 1)
        def _():
            wait_g(issue_g(g0 + 2, 0))
        for d in sd1:
            d.wait()
        return carry
    lax.fori_loop(0, _NG // 2, pair, 0)
    plsc.subcore_barrier()

    for k in range(_RPS // _ZR):
        r = row0 + k * _ZR
        pltpu.sync_copy(acc_sh.at[pl.ds(r, _ZR)], tmp_v)
        pltpu.sync_copy(tmp_v, out_hbm.at[c, pl.ds(r, _ZR)])


def _make_segsum():
    mesh = plsc.VectorSubcoreMesh(core_axis_name="c", subcore_axis_name="s",
                                  num_cores=_NC, num_subcores=_NS)
    return pl.kernel(
        _segsum_body,
        out_type=jax.ShapeDtypeStruct((_NC, _NP, _H), jnp.float32),
        mesh=mesh,
        scratch_types=[
            pltpu.VMEM((_NCH, _C), jnp.int32),
            pltpu.VMEM((_NCH, _C), jnp.int32),
            pltpu.VMEM((_NB, _C, _H), jnp.float32),
            pltpu.VMEM((_NB, _C, _H), jnp.float32),
            pltpu.VMEM_SHARED((_NP, _H), jnp.float32),
            pltpu.SemaphoreType.DMA,
            pltpu.SemaphoreType.DMA,
            pltpu.SemaphoreType.DMA,
        ],
        compiler_params=pltpu.CompilerParams(use_tc_tiling_on_sc=False),
    )


def _cnt_body(dst_hbm, out_hbm, dst_v, ones_v, tmp_v, acc_sh):
    c = lax.axis_index("c")
    s = lax.axis_index("s")
    wid = s * _NC + c

    for j in range(_CPS // 16):
        tmp_v[pl.ds(j * 16, 16)] = jnp.zeros((16,), jnp.float32)
    for j in range(_C // 16):
        ones_v[pl.ds(j * 16, 16)] = jnp.ones((16,), jnp.float32)
    pltpu.sync_copy(tmp_v, acc_sh.at[pl.ds(s * _CPS, _CPS)])
    pltpu.sync_copy(dst_hbm.at[wid], dst_v)
    plsc.subcore_barrier()

    def chunk(j, carry):
        pltpu.sync_copy(ones_v, acc_sh.at[dst_v.at[j]], add=True)
        return carry
    lax.fori_loop(0, _NCH, chunk, 0)
    plsc.subcore_barrier()

    pltpu.sync_copy(acc_sh.at[pl.ds(s * _CPS, _CPS)], tmp_v)
    pltpu.sync_copy(tmp_v, out_hbm.at[c, pl.ds(s * _CPS, _CPS)])


def _make_cnt():
    mesh = plsc.VectorSubcoreMesh(core_axis_name="c", subcore_axis_name="s",
                                  num_cores=_NC, num_subcores=_NS)
    return pl.kernel(
        _cnt_body,
        out_type=jax.ShapeDtypeStruct((_NC, _NP1), jnp.float32),
        mesh=mesh,
        scratch_types=[
            pltpu.VMEM((_NCH, _C), jnp.int32),
            pltpu.VMEM((_C,), jnp.float32),
            pltpu.VMEM((_CPS,), jnp.float32),
            pltpu.VMEM_SHARED((_NP1,), jnp.float32),
        ],
        compiler_params=pltpu.CompilerParams(use_tc_tiling_on_sc=False),
    )


def _full(shape):
    return pl.BlockSpec(shape, lambda i: tuple(0 for _ in shape))


def _pre_body(x_ref, wl_ref, wr_ref, b_ref, y_ref, r_ref):
    x = x_ref[...]
    y_ref[...] = jnp.dot(x, wl_ref[...], preferred_element_type=jnp.float32)
    r_ref[...] = jnp.dot(x, wr_ref[...], preferred_element_type=jnp.float32) + b_ref[...]


def _tc_pre(x, wl, wr, b):
    return pl.pallas_call(
        _pre_body,
        grid=(_G,),
        in_specs=[pl.BlockSpec((_R, _D), lambda i: (i, 0)),
                  _full((_D, _H)), _full((_D, _H)), _full((1, _H))],
        out_specs=[pl.BlockSpec((_R, _H), lambda i: (i, 0))] * 2,
        out_shape=[jax.ShapeDtypeStruct((_N, _H), jnp.float32)] * 2,
    )(x, wl, wr, b)


def _mid_body(p_ref, inv_ref, r_ref, wl_ref, wr_ref, b_ref, y_ref, r2_ref):
    agg = (p_ref[0] + p_ref[1]) * inv_ref[...]
    h = jnp.maximum(agg + r_ref[...], 0.0)
    y_ref[...] = jnp.dot(h, wl_ref[...], preferred_element_type=jnp.float32)
    r2_ref[...] = jnp.dot(h, wr_ref[...], preferred_element_type=jnp.float32) + b_ref[...]


def _tc_mid(p, inv, r, wl, wr, b):
    return pl.pallas_call(
        _mid_body,
        grid=(_G,),
        in_specs=[pl.BlockSpec((2, _R, _H), lambda i: (0, i, 0)),
                  pl.BlockSpec((_R, 1), lambda i: (i, 0)),
                  pl.BlockSpec((_R, _H), lambda i: (i, 0)),
                  _full((_H, _H)), _full((_H, _H)), _full((1, _H))],
        out_specs=[pl.BlockSpec((_R, _H), lambda i: (i, 0))] * 2,
        out_shape=[jax.ShapeDtypeStruct((_N, _H), jnp.float32)] * 2,
    )(p, inv, r, wl, wr, b)


def _mid2_body(p_ref, inv_ref, r_ref, wr3_ref, b3_ref, h_ref, rr_ref):
    agg = (p_ref[0] + p_ref[1]) * inv_ref[...]
    h = jnp.maximum(agg + r_ref[...], 0.0)
    h_ref[...] = h
    rr_ref[...] = jnp.dot(h, wr3_ref[...], preferred_element_type=jnp.float32) + b3_ref[...]


def _tc_mid2(p, inv, r, wr3, b3):
    return pl.pallas_call(
        _mid2_body,
        grid=(_G,),
        in_specs=[pl.BlockSpec((2, _R, _H), lambda i: (0, i, 0)),
                  pl.BlockSpec((_R, 1), lambda i: (i, 0)),
                  pl.BlockSpec((_R, _H), lambda i: (i, 0)),
                  _full((_H, _D)), _full((1, _D))],
        out_specs=[pl.BlockSpec((_R, _H), lambda i: (i, 0)),
                   pl.BlockSpec((_R, _D), lambda i: (i, 0))],
        out_shape=[jax.ShapeDtypeStruct((_N, _H), jnp.float32),
                   jax.ShapeDtypeStruct((_N, _D), jnp.float32)],
    )(p, inv, r, wr3, b3)


def _step_body(p_ref, inv_ref, rr_ref, xt_ref, dt_ref, wl3_ref,
               wl1_ref, wr1_ref, b1_ref, xn_ref, y_ref, r_ref):
    a3 = (p_ref[0] + p_ref[1]) * inv_ref[...]
    dx = jnp.dot(a3, wl3_ref[...], preferred_element_type=jnp.float32) + rr_ref[...]
    xn = xt_ref[...] + dt_ref[...] * dx
    xn_ref[...] = xn
    y_ref[...] = jnp.dot(xn, wl1_ref[...], preferred_element_type=jnp.float32)
    r_ref[...] = jnp.dot(xn, wr1_ref[...], preferred_element_type=jnp.float32) + b1_ref[...]


def _tc_step(p, inv, rr, xt, dt, wl3, wl1, wr1, b1):
    return pl.pallas_call(
        _step_body,
        grid=(_G,),
        in_specs=[pl.BlockSpec((2, _R, _H), lambda i: (0, i, 0)),
                  pl.BlockSpec((_R, 1), lambda i: (i, 0)),
                  pl.BlockSpec((_R, _D), lambda i: (i, 0)),
                  pl.BlockSpec((_R, _D), lambda i: (i, 0)),
                  _full((1, 1)), _full((_H, _D)),
                  _full((_D, _H)), _full((_D, _H)), _full((1, _H))],
        out_specs=[pl.BlockSpec((_R, _D), lambda i: (i, 0)),
                   pl.BlockSpec((_R, _H), lambda i: (i, 0)),
                   pl.BlockSpec((_R, _H), lambda i: (i, 0))],
        out_shape=[jax.ShapeDtypeStruct((_N, _D), jnp.float32),
                   jax.ShapeDtypeStruct((_N, _H), jnp.float32),
                   jax.ShapeDtypeStruct((_N, _H), jnp.float32)],
    )(p, inv, rr, xt, dt, wl3, wl1, wr1, b1)


def _dec_body(s_ref, w_ref, b_ref, o_ref):
    o_ref[...] = (jnp.dot(s_ref[0], w_ref[...], preferred_element_type=jnp.float32)
                  + b_ref[...])[None]


def _tc_dec(sol, w, b):
    return pl.pallas_call(
        _dec_body,
        grid=(_T, _G),
        in_specs=[pl.BlockSpec((1, _R, _D), lambda t, i: (t, i, 0)),
                  pl.BlockSpec((_D, 2), lambda t, i: (0, 0)),
                  pl.BlockSpec((1, 2), lambda t, i: (0, 0))],
        out_specs=pl.BlockSpec((1, _R, 2), lambda t, i: (t, i, 0)),
        out_shape=jax.ShapeDtypeStruct((_T, _N, 2), jnp.float32),
    )(sol, w, b)


def kernel(x, edge_index, time_span, Wl1, Wr1, b1, Wl2, Wr2, b2,
           Wl3, Wr3, b3, Wdec, bdec):
    padn = _EP - _E
    pad_i = jnp.arange(padn, dtype=jnp.int32)
    src3 = jnp.concatenate([edge_index[0], pad_i % _N]).reshape(_NW, _NCH, _C)
    dst3 = jnp.concatenate([edge_index[1], _N + pad_i % (_NP - _N)]
                           ).reshape(_NW, _NCH, _C)

    segsum = _make_segsum()
    cntk = _make_cnt()

    cntp = cntk(dst3)
    cnt = cntp[0, :_N] + cntp[1, :_N]
    inv = (1.0 / jnp.maximum(cnt, 1.0))[:, None]

    b1r = b1.reshape(1, _H)
    b2r = b2.reshape(1, _H)
    b3r = b3.reshape(1, _D)
    bdr = bdec.reshape(1, 2)
    dts = (time_span[1:] - time_span[:-1]).reshape(_T - 1, 1, 1)

    y1, r1 = _tc_pre(x, Wl1, Wr1, b1r)
    sols = [x]
    xt = x
    for i in range(1, _T):
        p1 = segsum(y1, src3, dst3)
        y2, r2 = _tc_mid(p1, inv, r1, Wl2, Wr2, b2r)
        p2 = segsum(y2, src3, dst3)
        h2, rr3 = _tc_mid2(p2, inv, r2, Wr3, b3r)
        p3 = segsum(h2, src3, dst3)
        xt, y1, r1 = _tc_step(p3, inv, rr3, xt, dts[i - 1], Wl3, Wl1, Wr1, b1r)
        sols.append(xt)

    solution = jnp.stack(sols, axis=0)
    trajectories = _tc_dec(solution, Wdec, bdr)
    return trajectories, solution
